# Initial kernel scaffold; baseline (speedup 1.0000x reference)
import jax
import jax.numpy as jnp
from jax.experimental import pallas as pl


def kernel(x, data, resolution):
    raise NotImplementedError("write your pallas kernel here")


# speedup vs baseline: 4.2750x; 4.2750x over previous
import functools

import jax
import jax.numpy as jnp
from jax import lax
from jax.experimental import pallas as pl
from jax.experimental.pallas import tpu as pltpu
from jax.experimental.pallas import tpu_sc as plsc

NC = 2
NS = 16
NW = NC * NS
L = 16

CHUNK = 2048
SUB = CHUNK // 128


def _vperm(vec, idx):
    return lax.gather(
        vec, idx[:, None],
        lax.GatherDimensionNumbers(
            offset_dims=(), collapsed_slice_dims=(0,), start_index_map=(0,)),
        (1,), mode=lax.GatherScatterMode.PROMISE_IN_BOUNDS)


def _sampler_kernel(Hs, Ws, n_per_w, x_hbm, tbl_hbm, out_hbm,
                    xbuf, wxbuf, wybuf, i00, i01, i10, i11,
                    t00, t01, t10, t11, obuf, sem):
    wid = lax.axis_index("s") * NC + lax.axis_index("c")
    base_w = wid * n_per_w
    n_chunks = n_per_w // CHUNK

    iota = lax.iota(jnp.int32, L)
    rep4 = iota // 4
    col4 = iota % 4
    cmask = col4 < 3
    zeros = jnp.zeros((L,), jnp.int32)
    ones = jnp.full((L,), 1, jnp.int32)
    wf = jnp.float32(Ws)
    hf = jnp.float32(Hs)

    def do_chunk(g, _):
        base = base_w + g * CHUNK

        pltpu.sync_copy(x_hbm.at[pl.ds(base, CHUNK), :], xbuf)

        def idx_body(t, _):
            rows = t * L + iota
            u = plsc.load_gather(xbuf, [rows, zeros])
            v = plsc.load_gather(xbuf, [rows, ones])
            xs = u * wf
            ys = v * hf
            xi = xs.astype(jnp.int32)
            yi = ys.astype(jnp.int32)
            fx = xs - xi.astype(jnp.float32)
            fy = ys - yi.astype(jnp.float32)
            x0 = jnp.minimum(jnp.maximum(xi, 0), Ws - 1)
            y0 = jnp.minimum(jnp.maximum(yi, 0), Hs - 1)
            x1 = jnp.minimum(x0 + 1, Ws - 1)
            y1 = jnp.minimum(y0 + 1, Hs - 1)
            r0 = y0 * Ws
            r1 = y1 * Ws
            rr = t >> 3
            cc = (t & 7) << 4
            i00[rr, pl.ds(cc, L)] = r0 + x0
            i01[rr, pl.ds(cc, L)] = r0 + x1
            i10[rr, pl.ds(cc, L)] = r1 + x0
            i11[rr, pl.ds(cc, L)] = r1 + x1
            wxbuf[pl.ds(t * L, L)] = fx
            wybuf[pl.ds(t * L, L)] = fy
            return 0

        lax.fori_loop(0, CHUNK // L, idx_body, 0, unroll=2)

        copies = []
        for ibuf, tbuf in ((i00, t00), (i01, t01), (i10, t10), (i11, t11)):
            for r in range(SUB):
                copies.append(pltpu.async_copy(
                    tbl_hbm.at[ibuf.at[r]],
                    tbuf.at[pl.ds(r * 128, 128), :], sem))
        for c in copies:
            c.wait()

        def blend_body(t, _):
            p0 = t * L
            wx = wxbuf[pl.ds(p0, L)]
            wy = wybuf[pl.ds(p0, L)]
            for j in range(4):
                prow = p0 + 4 * j + rep4
                wxr = plsc.load_gather(wxbuf, [prow])
                wyr = plsc.load_gather(wybuf, [prow])
                a00 = plsc.load_gather(t00, [prow, col4], mask=cmask)
                a01 = plsc.load_gather(t01, [prow, col4], mask=cmask)
                a10 = plsc.load_gather(t10, [prow, col4], mask=cmask)
                a11 = plsc.load_gather(t11, [prow, col4], mask=cmask)
                top = a00 + wxr * (a01 - a00)
                bot = a10 + wxr * (a11 - a10)
                o = top + wyr * (bot - top)
                plsc.store_scatter(obuf, [prow, col4], o, mask=cmask)
            return 0

        lax.fori_loop(0, CHUNK // L, blend_body, 0, unroll=2)

        pltpu.sync_copy(obuf, out_hbm.at[pl.ds(base, CHUNK), :])
        return 0

    lax.fori_loop(0, n_chunks, do_chunk, 0)


def kernel(x, data, resolution):
    del resolution
    Hs, Ws, C = data.shape
    N = x.shape[0]
    n_per_w = N // NW
    tbl = data.reshape(Hs * Ws, C).astype(jnp.float32)

    mesh = plsc.VectorSubcoreMesh(core_axis_name="c", subcore_axis_name="s")
    sampler = pl.kernel(
        functools.partial(_sampler_kernel, Hs, Ws, n_per_w),
        out_type=jax.ShapeDtypeStruct((N, C), jnp.float32),
        mesh=mesh,
        compiler_params=pltpu.CompilerParams(use_tc_tiling_on_sc=False, needs_layout_passes=False),
        scratch_types=[
            pltpu.VMEM((CHUNK, 2), jnp.float32),
            pltpu.VMEM((CHUNK,), jnp.float32),
            pltpu.VMEM((CHUNK,), jnp.float32),
            pltpu.VMEM((SUB, 128), jnp.int32),
            pltpu.VMEM((SUB, 128), jnp.int32),
            pltpu.VMEM((SUB, 128), jnp.int32),
            pltpu.VMEM((SUB, 128), jnp.int32),
            pltpu.VMEM((CHUNK, 3), jnp.float32),
            pltpu.VMEM((CHUNK, 3), jnp.float32),
            pltpu.VMEM((CHUNK, 3), jnp.float32),
            pltpu.VMEM((CHUNK, 3), jnp.float32),
            pltpu.VMEM((CHUNK, 3), jnp.float32),
            pltpu.SemaphoreType.DMA,
        ],
    )
    return sampler(x, tbl)


# speedup vs baseline: 28.9934x; 6.7821x over previous
import functools

import jax
import jax.numpy as jnp
from jax import lax
from jax.experimental import pallas as pl
from jax.experimental.pallas import tpu as pltpu
from jax.experimental.pallas import tpu_sc as plsc

NC = 2
NS = 16
NW = NC * NS
L = 16

CHUNK = 2048
SUB = CHUNK // 128


def _half_to_f32(h):
    sign = (h & 0x8000) << 16
    mag = (h & 0x7FFF) << 13
    f = plsc.bitcast(mag, jnp.float32) * jnp.float32(2.0 ** 112)
    return plsc.bitcast(plsc.bitcast(f, jnp.int32) | sign, jnp.float32)


def _sampler_kernel(Hs, Ws, n_per_w, x_hbm, lo_hbm, hi_hbm, out_hbm,
                    xbuf, wxbuf, wybuf, ibufs, lobufs, hibufs, obuf, sem):
    wid = lax.axis_index("s") * NC + lax.axis_index("c")
    base_w = wid * n_per_w
    n_chunks = n_per_w // CHUNK

    iota = lax.iota(jnp.int32, L)

    def do_chunk(g, _):
        base = base_w + g * CHUNK

        pltpu.sync_copy(x_hbm.at[pl.ds(2 * base, 2 * CHUNK)], xbuf)

        def idx_body(t, _):
            rows = (t * L + iota) * 2
            u = plsc.load_gather(xbuf, [rows])
            v = plsc.load_gather(xbuf, [rows + 1])
            xs = u * jnp.float32(Ws)
            ys = v * jnp.float32(Hs)
            xi = xs.astype(jnp.int32)
            yi = ys.astype(jnp.int32)
            fx = xs - xi.astype(jnp.float32)
            fy = ys - yi.astype(jnp.float32)
            x0 = jnp.minimum(jnp.maximum(xi, 0), Ws - 1)
            y0 = jnp.minimum(jnp.maximum(yi, 0), Hs - 1)
            x1 = jnp.minimum(x0 + 1, Ws - 1)
            y1 = jnp.minimum(y0 + 1, Hs - 1)
            r0 = y0 * Ws
            r1 = y1 * Ws
            rr = t >> 3
            cc = (t & 7) << 4
            ibufs[0][rr, pl.ds(cc, L)] = r0 + x0
            ibufs[1][rr, pl.ds(cc, L)] = r0 + x1
            ibufs[2][rr, pl.ds(cc, L)] = r1 + x0
            ibufs[3][rr, pl.ds(cc, L)] = r1 + x1
            wxbuf[pl.ds(t * L, L)] = fx
            wybuf[pl.ds(t * L, L)] = fy
            return 0

        lax.fori_loop(0, CHUNK // L, idx_body, 0, unroll=2)

        copies = []
        for k in range(4):
            for r in range(SUB):
                copies.append(pltpu.async_copy(
                    lo_hbm.at[ibufs[k].at[r]],
                    lobufs[k].at[pl.ds(r * 128, 128)], sem))
                copies.append(pltpu.async_copy(
                    hi_hbm.at[ibufs[k].at[r]],
                    hibufs[k].at[pl.ds(r * 128, 128)], sem))
        for c in copies:
            c.wait()

        def blend_body(t, _):
            p0 = t * L
            sl = pl.ds(p0, L)
            wx = wxbuf[sl]
            wy = wybuf[sl]
            tex = []
            for k in range(4):
                lov = lobufs[k][sl]
                c0 = _half_to_f32(lov & 0xFFFF)
                c1 = _half_to_f32(lax.shift_right_logical(lov, 16))
                tex.append((c0, c1, hibufs[k][sl]))
            opos = (p0 + iota) * 3
            for c in range(3):
                top = tex[0][c] + wx * (tex[1][c] - tex[0][c])
                bot = tex[2][c] + wx * (tex[3][c] - tex[2][c])
                o = top + wy * (bot - top)
                plsc.store_scatter(obuf, [opos + c], o)
            return 0

        lax.fori_loop(0, CHUNK // L, blend_body, 0, unroll=2)

        pltpu.sync_copy(obuf, out_hbm.at[pl.ds(3 * base, 3 * CHUNK)])
        return 0

    lax.fori_loop(0, n_chunks, do_chunk, 0)


def kernel(x, data, resolution):
    del resolution
    Hs, Ws, C = data.shape
    N = x.shape[0]
    n_per_w = N // NW

    lo = lax.bitcast_convert_type(data[:, :, :2], jnp.int32).reshape(Hs * Ws)
    hi = data[:, :, 2].astype(jnp.float32).reshape(Hs * Ws)
    xf = x.reshape(N * 2)

    mesh = plsc.VectorSubcoreMesh(core_axis_name="c", subcore_axis_name="s")
    sampler = pl.kernel(
        functools.partial(_sampler_kernel, Hs, Ws, n_per_w),
        out_type=jax.ShapeDtypeStruct((N * C,), jnp.float32),
        mesh=mesh,
        compiler_params=pltpu.CompilerParams(
            use_tc_tiling_on_sc=False, needs_layout_passes=False),
        scratch_types=[
            pltpu.VMEM((2 * CHUNK,), jnp.float32),
            pltpu.VMEM((CHUNK,), jnp.float32),
            pltpu.VMEM((CHUNK,), jnp.float32),
            [pltpu.VMEM((SUB, 128), jnp.int32)] * 4,
            [pltpu.VMEM((CHUNK,), jnp.int32)] * 4,
            [pltpu.VMEM((CHUNK,), jnp.float32)] * 4,
            pltpu.VMEM((3 * CHUNK,), jnp.float32),
            pltpu.SemaphoreType.DMA,
        ],
    )
    return sampler(xf, lo, hi).reshape(N, C)


# speedup vs baseline: 147.0059x; 5.0703x over previous
import functools

import jax
import jax.numpy as jnp
from jax import lax
from jax.experimental import pallas as pl
from jax.experimental.pallas import tpu as pltpu
from jax.experimental.pallas import tpu_sc as plsc

NC = 2
NS = 16
NW = NC * NS
L = 16

CHUNK = 2048
SUB = CHUNK // 128


def _half_to_f32(h):
    sign = (h & 0x8000) << 16
    mag = (h & 0x7FFF) << 13
    f = plsc.bitcast(mag, jnp.float32) * jnp.float32(2.0 ** 112)
    return plsc.bitcast(plsc.bitcast(f, jnp.int32) | sign, jnp.float32)


def _sampler_kernel(Hs, Ws, n_per_w, u_hbm, v_hbm, lo_hbm, hi_hbm,
                    o0_hbm, o1_hbm, o2_hbm,
                    ubuf, vbuf, wxbuf, wybuf, ibufs, lobufs, hibufs,
                    obufs, sem):
    wid = lax.axis_index("s") * NC + lax.axis_index("c")
    base_w = wid * n_per_w
    n_chunks = n_per_w // CHUNK

    def do_chunk(g, _):
        base = base_w + g * CHUNK

        pltpu.sync_copy(u_hbm.at[pl.ds(base, CHUNK)], ubuf)
        pltpu.sync_copy(v_hbm.at[pl.ds(base, CHUNK)], vbuf)

        def idx_body(t, _):
            sl = pl.ds(t * L, L)
            u = ubuf[sl]
            v = vbuf[sl]
            xs = u * jnp.float32(Ws)
            ys = v * jnp.float32(Hs)
            xi = xs.astype(jnp.int32)
            yi = ys.astype(jnp.int32)
            fx = xs - xi.astype(jnp.float32)
            fy = ys - yi.astype(jnp.float32)
            x0 = jnp.minimum(jnp.maximum(xi, 0), Ws - 1)
            y0 = jnp.minimum(jnp.maximum(yi, 0), Hs - 1)
            x1 = jnp.minimum(x0 + 1, Ws - 1)
            y1 = jnp.minimum(y0 + 1, Hs - 1)
            r0 = y0 * Ws
            r1 = y1 * Ws
            rr = t >> 3
            cc = (t & 7) << 4
            ibufs[0][rr, pl.ds(cc, L)] = r0 + x0
            ibufs[1][rr, pl.ds(cc, L)] = r0 + x1
            ibufs[2][rr, pl.ds(cc, L)] = r1 + x0
            ibufs[3][rr, pl.ds(cc, L)] = r1 + x1
            wxbuf[sl] = fx
            wybuf[sl] = fy
            return 0

        lax.fori_loop(0, CHUNK // L, idx_body, 0, unroll=2)

        copies = []
        for k in range(4):
            for r in range(SUB):
                copies.append(pltpu.async_copy(
                    lo_hbm.at[ibufs[k].at[r]],
                    lobufs[k].at[pl.ds(r * 128, 128)], sem))
                copies.append(pltpu.async_copy(
                    hi_hbm.at[ibufs[k].at[r]],
                    hibufs[k].at[pl.ds(r * 128, 128)], sem))
        for c in copies:
            c.wait()

        def blend_body(t, _):
            sl = pl.ds(t * L, L)
            wx = wxbuf[sl]
            wy = wybuf[sl]
            tex = []
            for k in range(4):
                lov = lobufs[k][sl]
                c0 = _half_to_f32(lov & 0xFFFF)
                c1 = _half_to_f32(lax.shift_right_logical(lov, 16))
                tex.append((c0, c1, hibufs[k][sl]))
            for c in range(3):
                top = tex[0][c] + wx * (tex[1][c] - tex[0][c])
                bot = tex[2][c] + wx * (tex[3][c] - tex[2][c])
                obufs[c][sl] = top + wy * (bot - top)
            return 0

        lax.fori_loop(0, CHUNK // L, blend_body, 0, unroll=2)

        pltpu.sync_copy(obufs[0], o0_hbm.at[pl.ds(base, CHUNK)])
        pltpu.sync_copy(obufs[1], o1_hbm.at[pl.ds(base, CHUNK)])
        pltpu.sync_copy(obufs[2], o2_hbm.at[pl.ds(base, CHUNK)])
        return 0

    lax.fori_loop(0, n_chunks, do_chunk, 0)


def kernel(x, data, resolution):
    del resolution
    Hs, Ws, C = data.shape
    N = x.shape[0]
    n_per_w = N // NW

    lo = lax.bitcast_convert_type(data[:, :, :2], jnp.int32).reshape(Hs * Ws)
    hi = data[:, :, 2].astype(jnp.float32).reshape(Hs * Ws)
    u = x[:, 0]
    v = x[:, 1]

    mesh = plsc.VectorSubcoreMesh(core_axis_name="c", subcore_axis_name="s")
    sampler = pl.kernel(
        functools.partial(_sampler_kernel, Hs, Ws, n_per_w),
        out_type=[jax.ShapeDtypeStruct((N,), jnp.float32)] * 3,
        mesh=mesh,
        compiler_params=pltpu.CompilerParams(
            use_tc_tiling_on_sc=False, needs_layout_passes=False),
        scratch_types=[
            pltpu.VMEM((CHUNK,), jnp.float32),
            pltpu.VMEM((CHUNK,), jnp.float32),
            pltpu.VMEM((CHUNK,), jnp.float32),
            pltpu.VMEM((CHUNK,), jnp.float32),
            [pltpu.VMEM((SUB, 128), jnp.int32)] * 4,
            [pltpu.VMEM((CHUNK,), jnp.int32)] * 4,
            [pltpu.VMEM((CHUNK,), jnp.float32)] * 4,
            [pltpu.VMEM((CHUNK,), jnp.float32)] * 3,
            pltpu.SemaphoreType.DMA,
        ],
    )
    o0, o1, o2 = sampler(u, v, lo, hi)
    return jnp.stack([o0, o1, o2], axis=1)


# speedup vs baseline: 147.4794x; 1.0032x over previous
import functools

import jax
import jax.numpy as jnp
from jax import lax
from jax.experimental import pallas as pl
from jax.experimental.pallas import tpu as pltpu
from jax.experimental.pallas import tpu_sc as plsc

NC = 2
NS = 16
NW = NC * NS
L = 16

CHUNK = 2048
SUB = CHUNK // 128


def _half_to_f32(h):
    sign = (h & 0x8000) << 16
    mag = (h & 0x7FFF) << 13
    f = plsc.bitcast(mag, jnp.float32) * jnp.float32(2.0 ** 112)
    return plsc.bitcast(plsc.bitcast(f, jnp.int32) | sign, jnp.float32)


def _sampler_kernel(Hs, Ws, n_per_w, u_hbm, v_hbm, lo_hbm, hi_hbm,
                    o0_hbm, o1_hbm, o2_hbm,
                    ubuf, vbuf, wxbuf, wybuf, ibufs, lobufs, hibufs,
                    obufs, sem):
    wid = lax.axis_index("s") * NC + lax.axis_index("c")
    base_w = wid * n_per_w
    n_chunks = n_per_w // CHUNK

    def do_chunk(g, _):
        base = base_w + g * CHUNK

        pltpu.sync_copy(u_hbm.at[pl.ds(base, CHUNK)], ubuf)
        pltpu.sync_copy(v_hbm.at[pl.ds(base, CHUNK)], vbuf)

        def idx_body(t, _):
            sl = pl.ds(t * L, L)
            u = ubuf[sl]
            v = vbuf[sl]
            xs = u * jnp.float32(Ws)
            ys = v * jnp.float32(Hs)
            xi = xs.astype(jnp.int32)
            yi = ys.astype(jnp.int32)
            fx = xs - xi.astype(jnp.float32)
            fy = ys - yi.astype(jnp.float32)
            x0 = jnp.minimum(jnp.maximum(xi, 0), Ws - 1)
            y0 = jnp.minimum(jnp.maximum(yi, 0), Hs - 1)
            x1 = jnp.minimum(x0 + 1, Ws - 1)
            y1 = jnp.minimum(y0 + 1, Hs - 1)
            r0 = y0 * Ws
            r1 = y1 * Ws
            ibufs[0][sl] = r0 + x0
            ibufs[1][sl] = r0 + x1
            ibufs[2][sl] = r1 + x0
            ibufs[3][sl] = r1 + x1
            wxbuf[sl] = fx
            wybuf[sl] = fy
            return 0

        lax.fori_loop(0, CHUNK // L, idx_body, 0, unroll=2)

        copies = []
        for k in range(4):
            copies.append(pltpu.async_copy(
                lo_hbm.at[ibufs[k]], lobufs[k], sem))
            copies.append(pltpu.async_copy(
                hi_hbm.at[ibufs[k]], hibufs[k], sem))
        for c in copies:
            c.wait()

        def blend_body(t, _):
            sl = pl.ds(t * L, L)
            wx = wxbuf[sl]
            wy = wybuf[sl]
            tex = []
            for k in range(4):
                lov = lobufs[k][sl]
                c0 = _half_to_f32(lov & 0xFFFF)
                c1 = _half_to_f32(lax.shift_right_logical(lov, 16))
                tex.append((c0, c1, hibufs[k][sl]))
            for c in range(3):
                top = tex[0][c] + wx * (tex[1][c] - tex[0][c])
                bot = tex[2][c] + wx * (tex[3][c] - tex[2][c])
                obufs[c][sl] = top + wy * (bot - top)
            return 0

        lax.fori_loop(0, CHUNK // L, blend_body, 0, unroll=2)

        pltpu.sync_copy(obufs[0], o0_hbm.at[pl.ds(base, CHUNK)])
        pltpu.sync_copy(obufs[1], o1_hbm.at[pl.ds(base, CHUNK)])
        pltpu.sync_copy(obufs[2], o2_hbm.at[pl.ds(base, CHUNK)])
        return 0

    lax.fori_loop(0, n_chunks, do_chunk, 0)


def kernel(x, data, resolution):
    del resolution
    Hs, Ws, C = data.shape
    N = x.shape[0]
    n_per_w = N // NW

    lo = lax.bitcast_convert_type(data[:, :, :2], jnp.int32).reshape(Hs * Ws)
    hi = data[:, :, 2].astype(jnp.float32).reshape(Hs * Ws)
    u = x[:, 0]
    v = x[:, 1]

    mesh = plsc.VectorSubcoreMesh(core_axis_name="c", subcore_axis_name="s")
    sampler = pl.kernel(
        functools.partial(_sampler_kernel, Hs, Ws, n_per_w),
        out_type=[jax.ShapeDtypeStruct((N,), jnp.float32)] * 3,
        mesh=mesh,
        compiler_params=pltpu.CompilerParams(
            use_tc_tiling_on_sc=False, needs_layout_passes=False),
        scratch_types=[
            pltpu.VMEM((CHUNK,), jnp.float32),
            pltpu.VMEM((CHUNK,), jnp.float32),
            pltpu.VMEM((CHUNK,), jnp.float32),
            pltpu.VMEM((CHUNK,), jnp.float32),
            [pltpu.VMEM((CHUNK,), jnp.int32)] * 4,
            [pltpu.VMEM((CHUNK,), jnp.int32)] * 4,
            [pltpu.VMEM((CHUNK,), jnp.float32)] * 4,
            [pltpu.VMEM((CHUNK,), jnp.float32)] * 3,
            pltpu.SemaphoreType.DMA,
        ],
    )
    o0, o1, o2 = sampler(u, v, lo, hi)
    return jnp.stack([o0, o1, o2], axis=1)


# speedup vs baseline: 175.1553x; 1.1877x over previous
import functools

import jax
import jax.numpy as jnp
from jax import lax
from jax.experimental import pallas as pl
from jax.experimental.pallas import tpu as pltpu
from jax.experimental.pallas import tpu_sc as plsc

NC = 2
NS = 16
NW = NC * NS
L = 16

CHUNK = 2048


def _half_to_f32(h):
    sign = (h & 0x8000) << 16
    mag = (h & 0x7FFF) << 13
    f = plsc.bitcast(mag, jnp.float32) * jnp.float32(2.0 ** 112)
    return plsc.bitcast(plsc.bitcast(f, jnp.int32) | sign, jnp.float32)


def _sampler_kernel(Hs, Ws, n_per_w, u_hbm, v_hbm, lo_hbm, hi_hbm,
                    o0_hbm, o1_hbm, o2_hbm,
                    ubuf, vbuf, wxbufs, wybufs, ibufs, lobufs, hibufs,
                    obufs, sems):
    wid = lax.axis_index("s") * NC + lax.axis_index("c")
    base_w = wid * n_per_w
    n_pairs = n_per_w // (2 * CHUNK)

    def stage_ab(base, s):
        pltpu.sync_copy(u_hbm.at[pl.ds(base, CHUNK)], ubuf)
        pltpu.sync_copy(v_hbm.at[pl.ds(base, CHUNK)], vbuf)

        def idx_body(t, _):
            sl = pl.ds(t * L, L)
            u = ubuf[sl]
            v = vbuf[sl]
            xs = u * jnp.float32(Ws)
            ys = v * jnp.float32(Hs)
            xi = xs.astype(jnp.int32)
            yi = ys.astype(jnp.int32)
            fx = xs - xi.astype(jnp.float32)
            fy = ys - yi.astype(jnp.float32)
            x0 = jnp.minimum(xi, Ws - 1)
            y0 = jnp.minimum(yi, Hs - 1)
            x1 = jnp.minimum(x0 + 1, Ws - 1)
            y1 = jnp.minimum(y0 + 1, Hs - 1)
            r0 = y0 * Ws
            r1 = y1 * Ws
            ibufs[s][0][sl] = r0 + x0
            ibufs[s][1][sl] = r0 + x1
            ibufs[s][2][sl] = r1 + x0
            ibufs[s][3][sl] = r1 + x1
            wxbufs[s][sl] = fx
            wybufs[s][sl] = fy
            return 0

        lax.fori_loop(0, CHUNK // L, idx_body, 0)

    def fire(s):
        for k in range(4):
            pltpu.async_copy(lo_hbm.at[ibufs[s][k]], lobufs[s][k], sems[s])
            pltpu.async_copy(hi_hbm.at[ibufs[s][k]], hibufs[s][k], sems[s])

    def drain(s):
        for k in range(4):
            pltpu.make_async_copy(
                lo_hbm.at[ibufs[s][k]], lobufs[s][k], sems[s]).wait()
            pltpu.make_async_copy(
                hi_hbm.at[ibufs[s][k]], hibufs[s][k], sems[s]).wait()

    def stage_de(base, s):
        def blend_body(t, _):
            sl = pl.ds(t * L, L)
            wx = wxbufs[s][sl]
            wy = wybufs[s][sl]
            tex = []
            for k in range(4):
                lov = lobufs[s][k][sl]
                c0 = _half_to_f32(lov & 0xFFFF)
                c1 = _half_to_f32(lax.shift_right_logical(lov, 16))
                tex.append((c0, c1, hibufs[s][k][sl]))
            for c in range(3):
                top = tex[0][c] + wx * (tex[1][c] - tex[0][c])
                bot = tex[2][c] + wx * (tex[3][c] - tex[2][c])
                obufs[c][sl] = top + wy * (bot - top)
            return 0

        lax.fori_loop(0, CHUNK // L, blend_body, 0)
        pltpu.sync_copy(obufs[0], o0_hbm.at[pl.ds(base, CHUNK)])
        pltpu.sync_copy(obufs[1], o1_hbm.at[pl.ds(base, CHUNK)])
        pltpu.sync_copy(obufs[2], o2_hbm.at[pl.ds(base, CHUNK)])

    stage_ab(base_w, 0)
    fire(0)

    def do_pair(gg, _):
        a = base_w + (2 * gg) * CHUNK
        b = a + CHUNK
        stage_ab(b, 1)
        fire(1)
        drain(0)
        stage_de(a, 0)

        @pl.when(gg < n_pairs - 1)
        def _():
            stage_ab(a + 2 * CHUNK, 0)
            fire(0)

        drain(1)
        stage_de(b, 1)
        return 0

    lax.fori_loop(0, n_pairs, do_pair, 0)


def kernel(x, data, resolution):
    del resolution
    Hs, Ws, C = data.shape
    N = x.shape[0]
    n_per_w = N // NW

    lo = lax.bitcast_convert_type(data[:, :, :2], jnp.int32).reshape(Hs * Ws)
    hi = data[:, :, 2].astype(jnp.float32).reshape(Hs * Ws)
    u = x[:, 0]
    v = x[:, 1]

    mesh = plsc.VectorSubcoreMesh(core_axis_name="c", subcore_axis_name="s")
    sampler = pl.kernel(
        functools.partial(_sampler_kernel, Hs, Ws, n_per_w),
        out_type=[jax.ShapeDtypeStruct((N,), jnp.float32)] * 3,
        mesh=mesh,
        compiler_params=pltpu.CompilerParams(
            use_tc_tiling_on_sc=False, needs_layout_passes=False),
        scratch_types=[
            pltpu.VMEM((CHUNK,), jnp.float32),
            pltpu.VMEM((CHUNK,), jnp.float32),
            [pltpu.VMEM((CHUNK,), jnp.float32)] * 2,
            [pltpu.VMEM((CHUNK,), jnp.float32)] * 2,
            [[pltpu.VMEM((CHUNK,), jnp.int32)] * 4] * 2,
            [[pltpu.VMEM((CHUNK,), jnp.int32)] * 4] * 2,
            [[pltpu.VMEM((CHUNK,), jnp.float32)] * 4] * 2,
            [pltpu.VMEM((CHUNK,), jnp.float32)] * 3,
            [pltpu.SemaphoreType.DMA] * 2,
        ],
    )
    o0, o1, o2 = sampler(u, v, lo, hi)
    return jnp.stack([o0, o1, o2], axis=1)


# speedup vs baseline: 186.2644x; 1.0634x over previous
import functools

import jax
import jax.numpy as jnp
from jax import lax
from jax.experimental import pallas as pl
from jax.experimental.pallas import tpu as pltpu
from jax.experimental.pallas import tpu_sc as plsc

NC = 2
NS = 16
NW = NC * NS
L = 16

CHUNK = 2048


def _half_to_f32(h):
    sign = (h & 0x8000) << 16
    mag = (h & 0x7FFF) << 13
    f = plsc.bitcast(mag, jnp.float32) * jnp.float32(2.0 ** 112)
    return plsc.bitcast(plsc.bitcast(f, jnp.int32) | sign, jnp.float32)


def _sampler_kernel(Hs, Ws, n_per_w, u_hbm, v_hbm, lo_hbm, cp_hbm,
                    o0_hbm, o1_hbm, o2_hbm,
                    ubuf, vbuf, wxbufs, wybufs, ibufs, lobufs, cpbufs,
                    obufs, sems):
    wid = lax.axis_index("s") * NC + lax.axis_index("c")
    base_w = wid * n_per_w
    n_pairs = n_per_w // (2 * CHUNK)

    def stage_ab(base, s):
        pltpu.sync_copy(u_hbm.at[pl.ds(base, CHUNK)], ubuf)
        pltpu.sync_copy(v_hbm.at[pl.ds(base, CHUNK)], vbuf)

        def idx_body(t, _):
            sl = pl.ds(t * L, L)
            u = ubuf[sl]
            v = vbuf[sl]
            xs = u * jnp.float32(Ws)
            ys = v * jnp.float32(Hs)
            xi = xs.astype(jnp.int32)
            yi = ys.astype(jnp.int32)
            fx = xs - xi.astype(jnp.float32)
            fy = ys - yi.astype(jnp.float32)
            x0 = jnp.minimum(xi, Ws - 1)
            y0 = jnp.minimum(yi, Hs - 1)
            x1 = jnp.minimum(x0 + 1, Ws - 1)
            y1 = jnp.minimum(y0 + 1, Hs - 1)
            r0 = y0 * Ws
            r1 = y1 * Ws
            ibufs[s][0][sl] = r0 + x0
            ibufs[s][1][sl] = r0 + x1
            ibufs[s][2][sl] = r1 + x0
            ibufs[s][3][sl] = r1 + x1
            wxbufs[s][sl] = fx
            wybufs[s][sl] = fy
            return 0

        lax.fori_loop(0, CHUNK // L, idx_body, 0)

    def fire(s):
        for k in range(4):
            pltpu.async_copy(lo_hbm.at[ibufs[s][k]], lobufs[s][k], sems[s])
        for k in range(2):
            pltpu.async_copy(cp_hbm.at[ibufs[s][2 * k]], cpbufs[s][k], sems[s])

    def drain(s):
        for k in range(4):
            pltpu.make_async_copy(
                lo_hbm.at[ibufs[s][k]], lobufs[s][k], sems[s]).wait()
        for k in range(2):
            pltpu.make_async_copy(
                cp_hbm.at[ibufs[s][2 * k]], cpbufs[s][k], sems[s]).wait()

    def stage_de(base, s):
        def blend_body(t, _):
            sl = pl.ds(t * L, L)
            wx = wxbufs[s][sl]
            wy = wybufs[s][sl]
            tex = []
            for k in range(4):
                lov = lobufs[s][k][sl]
                c0 = _half_to_f32(lov & 0xFFFF)
                c1 = _half_to_f32(lax.shift_right_logical(lov, 16))
                tex.append([c0, c1, None])
            for k in range(2):
                cpv = cpbufs[s][k][sl]
                tex[2 * k][2] = _half_to_f32(cpv & 0xFFFF)
                tex[2 * k + 1][2] = _half_to_f32(
                    lax.shift_right_logical(cpv, 16))
            for c in range(3):
                top = tex[0][c] + wx * (tex[1][c] - tex[0][c])
                bot = tex[2][c] + wx * (tex[3][c] - tex[2][c])
                obufs[c][sl] = top + wy * (bot - top)
            return 0

        lax.fori_loop(0, CHUNK // L, blend_body, 0)
        pltpu.sync_copy(obufs[0], o0_hbm.at[pl.ds(base, CHUNK)])
        pltpu.sync_copy(obufs[1], o1_hbm.at[pl.ds(base, CHUNK)])
        pltpu.sync_copy(obufs[2], o2_hbm.at[pl.ds(base, CHUNK)])

    stage_ab(base_w, 0)
    fire(0)

    def do_pair(gg, _):
        a = base_w + (2 * gg) * CHUNK
        b = a + CHUNK
        stage_ab(b, 1)
        fire(1)
        drain(0)
        stage_de(a, 0)

        @pl.when(gg < n_pairs - 1)
        def _():
            stage_ab(a + 2 * CHUNK, 0)
            fire(0)

        drain(1)
        stage_de(b, 1)
        return 0

    lax.fori_loop(0, n_pairs, do_pair, 0)


def kernel(x, data, resolution):
    del resolution
    Hs, Ws, C = data.shape
    N = x.shape[0]
    n_per_w = N // NW

    lo = lax.bitcast_convert_type(data[:, :, :2], jnp.int32).reshape(Hs * Ws)
    c2 = data[:, :, 2]
    c2n = jnp.concatenate([c2[:, 1:], c2[:, -1:]], axis=1)
    cp = lax.bitcast_convert_type(
        jnp.stack([c2, c2n], axis=-1), jnp.int32).reshape(Hs * Ws)
    u = x[:, 0]
    v = x[:, 1]

    mesh = plsc.VectorSubcoreMesh(core_axis_name="c", subcore_axis_name="s")
    sampler = pl.kernel(
        functools.partial(_sampler_kernel, Hs, Ws, n_per_w),
        out_type=[jax.ShapeDtypeStruct((N,), jnp.float32)] * 3,
        mesh=mesh,
        compiler_params=pltpu.CompilerParams(
            use_tc_tiling_on_sc=False, needs_layout_passes=False),
        scratch_types=[
            pltpu.VMEM((CHUNK,), jnp.float32),
            pltpu.VMEM((CHUNK,), jnp.float32),
            [pltpu.VMEM((CHUNK,), jnp.float32)] * 2,
            [pltpu.VMEM((CHUNK,), jnp.float32)] * 2,
            [[pltpu.VMEM((CHUNK,), jnp.int32)] * 4] * 2,
            [[pltpu.VMEM((CHUNK,), jnp.int32)] * 4] * 2,
            [[pltpu.VMEM((CHUNK,), jnp.int32)] * 2] * 2,
            [pltpu.VMEM((CHUNK,), jnp.float32)] * 3,
            [pltpu.SemaphoreType.DMA] * 2,
        ],
    )
    o0, o1, o2 = sampler(u, v, lo, cp)
    return jnp.stack([o0, o1, o2], axis=1)


# speedup vs baseline: 186.3449x; 1.0004x over previous
import functools

import jax
import jax.numpy as jnp
from jax import lax
from jax.experimental import pallas as pl
from jax.experimental.pallas import tpu as pltpu
from jax.experimental.pallas import tpu_sc as plsc

NC = 2
NS = 16
NW = NC * NS
L = 16

CHUNK = 4096


_F16_SCALE = 2.0 ** 112


def _pair_to_f32(lov):
    a = lax.shift_right_arithmetic(lax.shift_left(lov, 16), 3) & (-0x70002000)
    b = lax.shift_right_arithmetic(lov, 3) & (-0x70002000)
    lo = plsc.bitcast(a, jnp.float32) * jnp.float32(_F16_SCALE)
    hi = plsc.bitcast(b, jnp.float32) * jnp.float32(_F16_SCALE)
    return lo, hi


def _sampler_kernel(Hs, Ws, n_per_w, u_hbm, v_hbm, lo_hbm, cp_hbm,
                    o0_hbm, o1_hbm, o2_hbm,
                    ubuf, vbuf, wxbufs, wybufs, ibufs, lobufs, cpbufs,
                    obufs, sems):
    wid = lax.axis_index("s") * NC + lax.axis_index("c")
    base_w = wid * n_per_w
    n_pairs = n_per_w // (2 * CHUNK)

    def stage_ab(base, s):
        pltpu.sync_copy(u_hbm.at[pl.ds(base, CHUNK)], ubuf)
        pltpu.sync_copy(v_hbm.at[pl.ds(base, CHUNK)], vbuf)

        def idx_body(t, _):
            sl = pl.ds(t * L, L)
            u = ubuf[sl]
            v = vbuf[sl]
            xs = u * jnp.float32(Ws)
            ys = v * jnp.float32(Hs)
            xi = xs.astype(jnp.int32)
            yi = ys.astype(jnp.int32)
            fx = xs - xi.astype(jnp.float32)
            fy = ys - yi.astype(jnp.float32)
            x0 = jnp.minimum(xi, Ws - 1)
            y0 = jnp.minimum(yi, Hs - 1)
            x1 = jnp.minimum(x0 + 1, Ws - 1)
            y1 = jnp.minimum(y0 + 1, Hs - 1)
            r0 = y0 * Ws
            r1 = y1 * Ws
            ibufs[s][0][sl] = r0 + x0
            ibufs[s][1][sl] = r0 + x1
            ibufs[s][2][sl] = r1 + x0
            ibufs[s][3][sl] = r1 + x1
            wxbufs[s][sl] = fx
            wybufs[s][sl] = fy
            return 0

        lax.fori_loop(0, CHUNK // L, idx_body, 0)

    def fire(s):
        for k in range(4):
            pltpu.async_copy(lo_hbm.at[ibufs[s][k]], lobufs[s][k], sems[s])
        for k in range(2):
            pltpu.async_copy(cp_hbm.at[ibufs[s][2 * k]], cpbufs[s][k], sems[s])

    def drain(s):
        for k in range(4):
            pltpu.make_async_copy(
                lo_hbm.at[ibufs[s][k]], lobufs[s][k], sems[s]).wait()
        for k in range(2):
            pltpu.make_async_copy(
                cp_hbm.at[ibufs[s][2 * k]], cpbufs[s][k], sems[s]).wait()

    def stage_de(base, s):
        def blend_body(t, _):
            sl = pl.ds(t * L, L)
            wx = wxbufs[s][sl]
            wy = wybufs[s][sl]
            tex = []
            for k in range(4):
                c0, c1 = _pair_to_f32(lobufs[s][k][sl])
                tex.append([c0, c1, None])
            for k in range(2):
                c2a, c2b = _pair_to_f32(cpbufs[s][k][sl])
                tex[2 * k][2] = c2a
                tex[2 * k + 1][2] = c2b
            for c in range(3):
                top = tex[0][c] + wx * (tex[1][c] - tex[0][c])
                bot = tex[2][c] + wx * (tex[3][c] - tex[2][c])
                obufs[c][sl] = top + wy * (bot - top)
            return 0

        lax.fori_loop(0, CHUNK // L, blend_body, 0)
        pltpu.sync_copy(obufs[0], o0_hbm.at[pl.ds(base, CHUNK)])
        pltpu.sync_copy(obufs[1], o1_hbm.at[pl.ds(base, CHUNK)])
        pltpu.sync_copy(obufs[2], o2_hbm.at[pl.ds(base, CHUNK)])

    stage_ab(base_w, 0)
    fire(0)

    def do_pair(gg, _):
        a = base_w + (2 * gg) * CHUNK
        b = a + CHUNK
        stage_ab(b, 1)
        fire(1)
        drain(0)
        stage_de(a, 0)

        @pl.when(gg < n_pairs - 1)
        def _():
            stage_ab(a + 2 * CHUNK, 0)
            fire(0)

        drain(1)
        stage_de(b, 1)
        return 0

    lax.fori_loop(0, n_pairs, do_pair, 0)


def kernel(x, data, resolution):
    del resolution
    Hs, Ws, C = data.shape
    N = x.shape[0]
    n_per_w = N // NW

    lo = lax.bitcast_convert_type(data[:, :, :2], jnp.int32).reshape(Hs * Ws)
    c2 = data[:, :, 2]
    c2n = jnp.concatenate([c2[:, 1:], c2[:, -1:]], axis=1)
    cp = lax.bitcast_convert_type(
        jnp.stack([c2, c2n], axis=-1), jnp.int32).reshape(Hs * Ws)
    u = x[:, 0]
    v = x[:, 1]

    mesh = plsc.VectorSubcoreMesh(core_axis_name="c", subcore_axis_name="s")
    sampler = pl.kernel(
        functools.partial(_sampler_kernel, Hs, Ws, n_per_w),
        out_type=[jax.ShapeDtypeStruct((N,), jnp.float32)] * 3,
        mesh=mesh,
        compiler_params=pltpu.CompilerParams(
            use_tc_tiling_on_sc=False, needs_layout_passes=False),
        scratch_types=[
            pltpu.VMEM((CHUNK,), jnp.float32),
            pltpu.VMEM((CHUNK,), jnp.float32),
            [pltpu.VMEM((CHUNK,), jnp.float32)] * 2,
            [pltpu.VMEM((CHUNK,), jnp.float32)] * 2,
            [[pltpu.VMEM((CHUNK,), jnp.int32)] * 4] * 2,
            [[pltpu.VMEM((CHUNK,), jnp.int32)] * 4] * 2,
            [[pltpu.VMEM((CHUNK,), jnp.int32)] * 2] * 2,
            [pltpu.VMEM((CHUNK,), jnp.float32)] * 3,
            [pltpu.SemaphoreType.DMA] * 2,
        ],
    )
    o0, o1, o2 = sampler(u, v, lo, cp)
    return jnp.stack([o0, o1, o2], axis=1)
